# Initial kernel scaffold; baseline (speedup 1.0000x reference)
import jax
import jax.numpy as jnp
from jax.experimental import pallas as pl


def kernel(x, edge_index, edge_attr, W1, b1, W2, b2, gamma, beta):
    raise NotImplementedError("write your pallas kernel here")


# speedup vs baseline: 3.2716x; 3.2716x over previous
import functools

import jax
import jax.numpy as jnp
import numpy as np
from jax.experimental import pallas as pl
from jax.experimental.pallas import tpu as pltpu

_K = 16
_N = 10000
_D = 128
_NPAD = 10240
_ROWS = 128


def _knn_body(sqc_ref, x_ref, xt_ref, nbr_ref):
    xb = x_ref[...]
    dot = jax.lax.dot_general(
        xb, xt_ref[...], (((1,), (0,)), ((), ())),
        preferred_element_type=jnp.float32)
    sq_r = jnp.sum(xb * xb, axis=1, keepdims=True)
    d = sq_r - 2.0 * dot + sqc_ref[...]
    i = pl.program_id(0)
    rows = jax.lax.broadcasted_iota(jnp.int32, d.shape, 0) + i * _ROWS
    cols = jax.lax.broadcasted_iota(jnp.int32, d.shape, 1)
    inf = jnp.float32(jnp.inf)
    d = jnp.where(cols == rows, inf, d)
    big = jnp.int32(2**30)
    idxs = []
    for _ in range(_K):
        m = jnp.min(d, axis=1, keepdims=True)
        idx = jnp.min(jnp.where(d == m, cols, big), axis=1, keepdims=True)
        idxs.append(idx)
        d = jnp.where(cols == idx, inf, d)
    nbr_ref[...] = jnp.concatenate(idxs, axis=1)


def _knn(x, sq):
    sqc = jnp.concatenate(
        [sq, jnp.full((_NPAD - _N,), jnp.inf, jnp.float32)])[None, :]
    xt = jnp.pad(x.T, ((0, 0), (0, _NPAD - _N)))
    grid = (pl.cdiv(_N, _ROWS),)
    return pl.pallas_call(
        _knn_body,
        grid=grid,
        in_specs=[
            pl.BlockSpec((1, _NPAD), lambda i: (0, 0)),
            pl.BlockSpec((_ROWS, _D), lambda i: (i, 0)),
            pl.BlockSpec((_D, _NPAD), lambda i: (0, 0)),
        ],
        out_specs=pl.BlockSpec((_ROWS, _K), lambda i: (i, 0)),
        out_shape=jax.ShapeDtypeStruct((_N, _K), jnp.int32),
    )(sqc, x, xt)


def kernel(x, edge_index, edge_attr, W1, b1, W2, b2, gamma, beta):
    sq = jnp.sum(x * x, axis=1)
    nbr = _knn(x, sq)
    dst = jnp.repeat(jnp.arange(_N), _K)
    src = nbr.reshape(-1)
    src_u = jnp.concatenate([src, dst])
    dst_u = jnp.concatenate([dst, src])
    ei = jnp.stack([src_u, dst_u])

    xi = x[dst_u]
    xj = x[src_u]
    m = jnp.concatenate([xi, xj - xi], axis=1)
    h = jnp.maximum(m @ W1 + b1, 0.0) @ W2 + b2
    out = jax.ops.segment_max(h, dst_u, num_segments=_N)
    mu = jnp.mean(out, axis=0)
    var = jnp.var(out, axis=0)
    out = (out - mu) / jnp.sqrt(var + 1e-5) * gamma + beta
    return (out, ei, edge_attr)


# speedup vs baseline: 4.7258x; 1.4445x over previous
import functools

import jax
import jax.numpy as jnp
import numpy as np
from jax.experimental import pallas as pl
from jax.experimental.pallas import tpu as pltpu

_K = 16
_N = 10000
_D = 128
_NPAD = 10240
_ROWS = 128


def _knn_body(sqc_ref, x_ref, xt_ref, nbr_ref):
    xb = x_ref[...]
    dot = jax.lax.dot_general(
        xb, xt_ref[...], (((1,), (0,)), ((), ())),
        preferred_element_type=jnp.float32)
    sq_r = jnp.sum(xb * xb, axis=1, keepdims=True)
    d = sq_r - 2.0 * dot + sqc_ref[...]
    i = pl.program_id(0)
    rows = jax.lax.broadcasted_iota(jnp.int32, d.shape, 0) + i * _ROWS
    cols = jax.lax.broadcasted_iota(jnp.int32, d.shape, 1)
    inf = jnp.float32(jnp.inf)
    d = jnp.where(cols == rows, inf, d)
    big = jnp.int32(2**30)
    idxs = []
    for _ in range(_K):
        m = jnp.min(d, axis=1, keepdims=True)
        idx = jnp.min(jnp.where(d == m, cols, big), axis=1, keepdims=True)
        idxs.append(idx)
        d = jnp.where(cols == idx, inf, d)
    nbr_ref[...] = jnp.concatenate(idxs, axis=1)


def _knn(x, sq):
    sqc = jnp.concatenate(
        [sq, jnp.full((_NPAD - _N,), jnp.inf, jnp.float32)])[None, :]
    xt = jnp.pad(x.T, ((0, 0), (0, _NPAD - _N)))
    grid = (pl.cdiv(_N, _ROWS),)
    return pl.pallas_call(
        _knn_body,
        grid=grid,
        in_specs=[
            pl.BlockSpec((1, _NPAD), lambda i: (0, 0)),
            pl.BlockSpec((_ROWS, _D), lambda i: (i, 0)),
            pl.BlockSpec((_D, _NPAD), lambda i: (0, 0)),
        ],
        out_specs=pl.BlockSpec((_ROWS, _K), lambda i: (i, 0)),
        out_shape=jax.ShapeDtypeStruct((_N, _K), jnp.int32),
    )(sqc, x, xt)


_NB = 256


def _tables_body(x_ref, wd_ref, wb_ref, b1_ref, c_ref, b_ref):
    xb = x_ref[...]
    c_ref[...] = jax.lax.dot_general(
        xb, wd_ref[...], (((1,), (0,)), ((), ())),
        preferred_element_type=jnp.float32) + b1_ref[...]
    b_ref[...] = jax.lax.dot_general(
        xb, wb_ref[...], (((1,), (0,)), ((), ())),
        preferred_element_type=jnp.float32)


def _tables(x, W1, b1):
    wdiff = W1[:_D] - W1[_D:]
    wb = W1[_D:]
    rb = 1024
    return pl.pallas_call(
        _tables_body,
        grid=(pl.cdiv(_N, rb),),
        in_specs=[
            pl.BlockSpec((rb, _D), lambda i: (i, 0)),
            pl.BlockSpec((_D, _D), lambda i: (0, 0)),
            pl.BlockSpec((_D, _D), lambda i: (0, 0)),
            pl.BlockSpec((1, _D), lambda i: (0, 0)),
        ],
        out_specs=[
            pl.BlockSpec((rb, _D), lambda i: (i, 0)),
            pl.BlockSpec((rb, _D), lambda i: (i, 0)),
        ],
        out_shape=[
            jax.ShapeDtypeStruct((_N, _D), jnp.float32),
            jax.ShapeDtypeStruct((_N, _D), jnp.float32),
        ],
    )(x, wdiff, wb, b1[None, :])


def _edge_mlp_body(c_ref, b_ref, gc_ref, gb_ref, w2_ref, fwd_ref, rev_ref):
    cb = c_ref[...]
    bb = b_ref[...]
    w2 = w2_ref[...]
    pres = []
    for t in range(_K):
        pres.append(cb + gb_ref[t])
    for t in range(_K):
        pres.append(gc_ref[t] + bb)
    h = jax.lax.dot_general(
        jnp.maximum(jnp.concatenate(pres, axis=0), 0.0), w2,
        (((1,), (0,)), ((), ())), preferred_element_type=jnp.float32)
    acc = h[0:_NB]
    for t in range(1, _K):
        acc = jnp.maximum(acc, h[t * _NB:(t + 1) * _NB])
    fwd_ref[...] = acc
    for t in range(_K):
        rev_ref[t] = h[(_K + t) * _NB:(_K + t + 1) * _NB]


def _edge_mlp(C, B, GC, GB, W2):
    return pl.pallas_call(
        _edge_mlp_body,
        grid=(pl.cdiv(_N, _NB),),
        in_specs=[
            pl.BlockSpec((_NB, _D), lambda i: (i, 0)),
            pl.BlockSpec((_NB, _D), lambda i: (i, 0)),
            pl.BlockSpec((_K, _NB, _D), lambda i: (0, i, 0)),
            pl.BlockSpec((_K, _NB, _D), lambda i: (0, i, 0)),
            pl.BlockSpec((_D, _D), lambda i: (0, 0)),
        ],
        out_specs=[
            pl.BlockSpec((_NB, _D), lambda i: (i, 0)),
            pl.BlockSpec((_K, _NB, _D), lambda i: (0, i, 0)),
        ],
        out_shape=[
            jax.ShapeDtypeStruct((_N, _D), jnp.float32),
            jax.ShapeDtypeStruct((_K, _N, _D), jnp.float32),
        ],
    )(C, B, GC, GB, W2)


def _bn_body(o_ref, g_ref, be_ref, b2_ref, out_ref):
    o = o_ref[...] + b2_ref[...]
    mu = jnp.mean(o, axis=0, keepdims=True)
    var = jnp.mean((o - mu) * (o - mu), axis=0, keepdims=True)
    out_ref[...] = (o - mu) * jax.lax.rsqrt(var + 1e-5) * g_ref[...] + be_ref[...]


def _batchnorm(o, gamma, beta, b2):
    return pl.pallas_call(
        _bn_body,
        in_specs=[
            pl.BlockSpec((_N, _D), lambda: (0, 0)),
            pl.BlockSpec((1, _D), lambda: (0, 0)),
            pl.BlockSpec((1, _D), lambda: (0, 0)),
            pl.BlockSpec((1, _D), lambda: (0, 0)),
        ],
        out_specs=pl.BlockSpec((_N, _D), lambda: (0, 0)),
        out_shape=jax.ShapeDtypeStruct((_N, _D), jnp.float32),
    )(o, gamma[None, :], beta[None, :], b2[None, :])


def kernel(x, edge_index, edge_attr, W1, b1, W2, b2, gamma, beta):
    sq = jnp.sum(x * x, axis=1)
    nbr = _knn(x, sq)
    dst = jnp.repeat(jnp.arange(_N), _K)
    src = nbr.reshape(-1)
    src_u = jnp.concatenate([src, dst])
    dst_u = jnp.concatenate([dst, src])
    ei = jnp.stack([src_u, dst_u])

    C, B = _tables(x, W1, b1)
    nbr_t = nbr.T
    GC = C[nbr_t]
    GB = B[nbr_t]
    out_fwd, h_rev = _edge_mlp(C, B, GC, GB, W2)
    seg = jax.ops.segment_max(
        h_rev.reshape(_K * _N, _D), nbr_t.reshape(-1), num_segments=_N)
    o = jnp.maximum(out_fwd, seg)
    out = _batchnorm(o, gamma, beta, b2)
    return (out, ei, edge_attr)
